# baseline (speedup 1.0000x reference)
import functools

import jax
import jax.numpy as jnp
from jax import lax
from jax.experimental import pallas as pl
from jax.experimental.pallas import tpu as pltpu
from jax.experimental.pallas import tpu_sc as plsc

B, L = 4096, 200
N = B * L
NUM_V = 1000000
TOKEN_D = 64
OUT_D = 112
PAD_D = 128
NC, NS = 2, 16
NW = NC * NS
ROWS_PER_W = N // NW
C = 200
N_CHUNKS = ROWS_PER_W // C
G_SPLITS = ((0, 128), (128, 72))


def _body(tok_hbm, cmb_hbm,
          tokw_hbm, wkw_hbm, hrw_hbm, duw_hbm,
          out_hbm,
          ti0, ti1, ci0, ci1,
          wtab_v, htab_v, dtab_v,
          tr0, tr1, o0, o1, gsem, osem):
    out_hbm = out_hbm.reshape(N, OUT_D)
    wid = lax.axis_index("s") * NC + lax.axis_index("c")
    iota = jax.lax.iota(jnp.int32, 16)
    w0 = wid * ROWS_PER_W

    pltpu.sync_copy(wkw_hbm, wtab_v)
    pltpu.sync_copy(hrw_hbm, htab_v)
    pltpu.sync_copy(duw_hbm, dtab_v)

    def fire_gathers(c, ti, ci):
        base = w0 + c * C
        pltpu.sync_copy(tok_hbm.at[pl.ds(base, C)], ti)
        pltpu.sync_copy(cmb_hbm.at[pl.ds(base, C)], ci)
        tr = tr0 if ti is ti0 else tr1
        for off, g in G_SPLITS:
            pltpu.async_copy(
                tokw_hbm.at[ti.at[pl.ds(off, g)]],
                tr.at[pl.ds(off, g)],
                gsem)

    def wait_gathers(tr):
        for off, g in G_SPLITS:
            pltpu.make_async_copy(
                tokw_hbm.at[pl.ds(0, g)],
                tr.at[pl.ds(off, g)],
                gsem).wait()

    def wait_out(o):
        pltpu.make_async_copy(o, out_hbm.at[pl.ds(0, C)], osem).wait()

    fire_gathers(0, ti0, ci0)

    def step(c, ti_n, ci_n, tr, ci, o):
        base = w0 + c * C

        @pl.when(c >= 2)
        def _():
            wait_out(o)

        wait_gathers(tr)

        def row_body(i, carry2):
            bidx = jnp.full((16,), i, jnp.int32)
            cb = plsc.load_gather(ci, [bidx])
            for k in range(4):
                o[i, pl.ds(k * 16, 16)] = tr[i, pl.ds(k * 16, 16)]
            o[i, pl.ds(64, 16)] = plsc.load_gather(
                wtab_v, [lax.shift_right_logical(cb, 18) + iota])
            o[i, pl.ds(80, 16)] = plsc.load_gather(
                htab_v, [(lax.shift_right_logical(cb, 9) & 511) + iota])
            o[i, pl.ds(96, 16)] = plsc.load_gather(
                dtab_v, [(cb & 511) + iota])
            return carry2
        lax.fori_loop(0, C, row_body, 0, unroll=2)

        pltpu.async_copy(o, out_hbm.at[pl.ds(base, C)], osem)

        @pl.when(c + 1 < N_CHUNKS)
        def _():
            fire_gathers(c + 1, ti_n, ci_n)

    def pair_body(h, carry):
        step(2 * h, ti1, ci1, tr0, ci0, o0)
        step(2 * h + 1, ti0, ci0, tr1, ci1, o1)
        return carry

    lax.fori_loop(0, N_CHUNKS // 2, pair_body, 0, unroll=False)

    wait_out(o0)
    wait_out(o1)


@jax.jit
def _launch(tok, cmb, tokw, wkw, hrw, duw):
    mesh = plsc.VectorSubcoreMesh(core_axis_name="c", subcore_axis_name="s")
    kfn = functools.partial(
        pl.kernel,
        mesh=mesh,
        compiler_params=pltpu.CompilerParams(needs_layout_passes=False),
        out_type=jax.ShapeDtypeStruct((B, L, OUT_D), jnp.float32),
        scratch_types=[
            pltpu.VMEM((C,), jnp.int32),
            pltpu.VMEM((C,), jnp.int32),
            pltpu.VMEM((C,), jnp.int32),
            pltpu.VMEM((C,), jnp.int32),
            pltpu.VMEM((7 * 16,), jnp.float32),
            pltpu.VMEM((24 * 16,), jnp.float32),
            pltpu.VMEM((24 * 16,), jnp.float32),
            pltpu.VMEM((C, PAD_D), jnp.float32),
            pltpu.VMEM((C, PAD_D), jnp.float32),
            pltpu.VMEM((C, OUT_D), jnp.float32),
            pltpu.VMEM((C, OUT_D), jnp.float32),
            pltpu.SemaphoreType.DMA,
            pltpu.SemaphoreType.DMA,
        ],
    )(_body)
    return kfn(tok, cmb, tokw, wkw, hrw, duw)


def kernel(token, week, hour, duration, token_w, week_w, hour_w, dur_w):
    token_w = jnp.pad(token_w, ((0, 0), (0, PAD_D - TOKEN_D)))
    tok = token.reshape(-1).astype(jnp.int32)
    cmb = ((week.astype(jnp.int32) << 22)
           | (hour.astype(jnp.int32) << 13)
           | (duration.astype(jnp.int32) << 4)).reshape(-1)
    return _launch(tok, cmb, token_w,
                   week_w.reshape(-1), hour_w.reshape(-1), dur_w.reshape(-1))


# speedup vs baseline: 1.1764x; 1.1764x over previous
import functools

import jax
import jax.numpy as jnp
from jax import lax
from jax.experimental import pallas as pl
from jax.experimental.pallas import tpu as pltpu
from jax.experimental.pallas import tpu_sc as plsc
from jax.experimental.layout import Layout as _Layout
from jax.experimental.pallas import tpu as _pltpu_unused
from jax.experimental import layout as _layout_mod

B, L = 4096, 200
N = B * L
NUM_V = 1000000
TOKEN_D = 64
OUT_D = 112
PAD_D = 128
NC, NS = 2, 16
NW = NC * NS
ROWS_PER_W = N // NW
C = 200
N_CHUNKS = ROWS_PER_W // C
G_SPLITS = ((0, 128), (128, 72))


def _body(tok_hbm, cmb_hbm,
          tokw_hbm, wkw_hbm, hrw_hbm, duw_hbm,
          out_hbm,
          ti0, ti1, ci0, ci1,
          wtab_v, htab_v, dtab_v,
          tr0, tr1, o0, o1, gsem, osem, isem):
    out_hbm = out_hbm.reshape(N, OUT_D)
    wid = lax.axis_index("s") * NC + lax.axis_index("c")
    iota = jax.lax.iota(jnp.int32, 16)
    w0 = wid * ROWS_PER_W

    pltpu.sync_copy(wkw_hbm, wtab_v)
    pltpu.sync_copy(hrw_hbm, htab_v)
    pltpu.sync_copy(duw_hbm, dtab_v)

    def fire_idx(c, ti, ci):
        base = w0 + c * C
        pltpu.async_copy(tok_hbm.at[pl.ds(base, C)], ti, isem)
        pltpu.async_copy(cmb_hbm.at[pl.ds(base, C)], ci, isem)

    def wait_idx(ti, ci):
        pltpu.make_async_copy(tok_hbm.at[pl.ds(0, C)], ti, isem).wait()
        pltpu.make_async_copy(cmb_hbm.at[pl.ds(0, C)], ci, isem).wait()

    def fire_gathers(ti, tr):
        for off, g in G_SPLITS:
            pltpu.async_copy(
                tokw_hbm.at[ti.at[pl.ds(off, g)]],
                tr.at[pl.ds(off, g)],
                gsem)

    def wait_gathers(tr):
        for off, g in G_SPLITS:
            pltpu.make_async_copy(
                tokw_hbm.at[pl.ds(0, g)],
                tr.at[pl.ds(off, g)],
                gsem).wait()

    def wait_out(o):
        pltpu.make_async_copy(o, out_hbm.at[pl.ds(0, C)], osem).wait()

    fire_idx(0, ti0, ci0)
    wait_idx(ti0, ci0)
    fire_gathers(ti0, tr0)
    fire_idx(1, ti1, ci1)

    def step(c, ti_n, ci_n, ti, tr_n, tr, ci, o):
        base = w0 + c * C

        @pl.when(c >= 2)
        def _():
            wait_out(o)

        wait_gathers(tr)

        @pl.when(c + 1 < N_CHUNKS)
        def _():
            wait_idx(ti_n, ci_n)
            fire_gathers(ti_n, tr_n)

        def row_body(i, carry2):
            bidx = jnp.full((16,), i, jnp.int32)
            cb = plsc.load_gather(ci, [bidx])
            for k in range(4):
                o[i, pl.ds(k * 16, 16)] = tr[i, pl.ds(k * 16, 16)]
            o[i, pl.ds(64, 16)] = plsc.load_gather(
                wtab_v, [lax.shift_right_logical(cb, 18) + iota])
            o[i, pl.ds(80, 16)] = plsc.load_gather(
                htab_v, [(lax.shift_right_logical(cb, 9) & 511) + iota])
            o[i, pl.ds(96, 16)] = plsc.load_gather(
                dtab_v, [(cb & 511) + iota])
            return carry2
        lax.fori_loop(0, C, row_body, 0, unroll=2)

        pltpu.async_copy(o, out_hbm.at[pl.ds(base, C)], osem)

        @pl.when(c + 2 < N_CHUNKS)
        def _():
            fire_idx(c + 2, ti, ci)

    def pair_body(h, carry):
        step(2 * h, ti1, ci1, ti0, tr1, tr0, ci0, o0)
        step(2 * h + 1, ti0, ci0, ti1, tr0, tr1, ci1, o1)
        return carry

    lax.fori_loop(0, N_CHUNKS // 2, pair_body, 0, unroll=False)

    wait_out(o0)
    wait_out(o1)


@jax.jit
def _launch(tok, cmb, tokw, wkw, hrw, duw):
    mesh = plsc.VectorSubcoreMesh(core_axis_name="c", subcore_axis_name="s")
    kfn = functools.partial(
        pl.kernel,
        mesh=mesh,
        compiler_params=pltpu.CompilerParams(needs_layout_passes=False),
        out_type=jax.ShapeDtypeStruct((B, L, OUT_D), jnp.float32),
        scratch_types=[
            pltpu.VMEM((C,), jnp.int32),
            pltpu.VMEM((C,), jnp.int32),
            pltpu.VMEM((C,), jnp.int32),
            pltpu.VMEM((C,), jnp.int32),
            pltpu.VMEM((7 * 16,), jnp.float32),
            pltpu.VMEM((24 * 16,), jnp.float32),
            pltpu.VMEM((24 * 16,), jnp.float32),
            pltpu.VMEM((C, PAD_D), jnp.float32),
            pltpu.VMEM((C, PAD_D), jnp.float32),
            pltpu.VMEM((C, OUT_D), jnp.float32),
            pltpu.VMEM((C, OUT_D), jnp.float32),
            pltpu.SemaphoreType.DMA,
            pltpu.SemaphoreType.DMA,
            pltpu.SemaphoreType.DMA,
        ],
    )(_body)
    return kfn(tok, cmb, tokw, wkw, hrw, duw)


def kernel(token, week, hour, duration, token_w, week_w, hour_w, dur_w):
    token_w = _layout_mod.with_layout_constraint(token_w, _Layout((1, 0)))
    token_w = jnp.pad(token_w, ((0, 0), (0, PAD_D - TOKEN_D)))
    tok = token.reshape(-1).astype(jnp.int32)
    cmb = ((week.astype(jnp.int32) << 22)
           | (hour.astype(jnp.int32) << 13)
           | (duration.astype(jnp.int32) << 4)).reshape(-1)
    return _launch(tok, cmb, token_w,
                   week_w.reshape(-1), hour_w.reshape(-1), dur_w.reshape(-1))


# speedup vs baseline: 1.1858x; 1.0080x over previous
import functools

import jax
import jax.numpy as jnp
from jax import lax
from jax.experimental import pallas as pl
from jax.experimental.pallas import tpu as pltpu
from jax.experimental.pallas import tpu_sc as plsc

B, L = 4096, 200
N = B * L
NUM_V = 1000000
TOKEN_D = 64
OUT_D = 112
PAD_D = 128
NC, NS = 2, 16
NW = NC * NS
ROWS_PER_W = N // NW
C = 200
N_CHUNKS = ROWS_PER_W // C
G_SPLITS = ((0, 128), (128, 72))


def _body(tok_hbm, cmb_hbm,
          tokw_hbm, wkw_hbm, hrw_hbm, duw_hbm,
          out_hbm,
          ti0, ti1, ci0, ci1,
          wtab_v, htab_v, dtab_v,
          tr0, tr1, o0, o1, gsem, osem, isem):
    out_hbm = out_hbm.reshape(N, OUT_D)
    wid = lax.axis_index("s") * NC + lax.axis_index("c")
    iota = jax.lax.iota(jnp.int32, 16)
    w0 = wid * ROWS_PER_W

    pltpu.sync_copy(wkw_hbm, wtab_v)
    pltpu.sync_copy(hrw_hbm, htab_v)
    pltpu.sync_copy(duw_hbm, dtab_v)

    def fire_idx(c, ti, ci):
        base = w0 + c * C
        pltpu.async_copy(tok_hbm.at[pl.ds(base, C)], ti, isem)
        pltpu.async_copy(cmb_hbm.at[pl.ds(base, C)], ci, isem)

    def wait_idx(ti, ci):
        pltpu.make_async_copy(tok_hbm.at[pl.ds(0, C)], ti, isem).wait()
        pltpu.make_async_copy(cmb_hbm.at[pl.ds(0, C)], ci, isem).wait()

    def fire_gathers(ti, tr):
        for off, g in G_SPLITS:
            pltpu.async_copy(
                tokw_hbm.at[ti.at[pl.ds(off, g)]],
                tr.at[pl.ds(off, g)],
                gsem)

    def wait_gathers(tr):
        for off, g in G_SPLITS:
            pltpu.make_async_copy(
                tokw_hbm.at[pl.ds(0, g)],
                tr.at[pl.ds(off, g)],
                gsem).wait()

    def wait_out(o):
        pltpu.make_async_copy(o, out_hbm.at[pl.ds(0, C)], osem).wait()

    fire_idx(0, ti0, ci0)
    wait_idx(ti0, ci0)
    fire_gathers(ti0, tr0)
    fire_idx(1, ti1, ci1)

    def step(c, ti_n, ci_n, ti, tr_n, tr, ci, o):
        base = w0 + c * C

        @pl.when(c >= 2)
        def _():
            wait_out(o)

        wait_gathers(tr)

        @pl.when(c + 1 < N_CHUNKS)
        def _():
            wait_idx(ti_n, ci_n)
            fire_gathers(ti_n, tr_n)

        def row_body(i, carry2):
            bidx = jnp.full((16,), i, jnp.int32)
            cb = plsc.load_gather(ci, [bidx])
            for k in range(4):
                o[i, pl.ds(k * 16, 16)] = tr[i, pl.ds(k * 16, 16)]
            o[i, pl.ds(64, 16)] = plsc.load_gather(
                wtab_v, [lax.shift_right_logical(cb, 18) + iota])
            o[i, pl.ds(80, 16)] = plsc.load_gather(
                htab_v, [(lax.shift_right_logical(cb, 9) & 511) + iota])
            o[i, pl.ds(96, 16)] = plsc.load_gather(
                dtab_v, [(cb & 511) + iota])
            return carry2
        lax.fori_loop(0, C, row_body, 0, unroll=4)

        pltpu.async_copy(o, out_hbm.at[pl.ds(base, C)], osem)

        @pl.when(c + 2 < N_CHUNKS)
        def _():
            fire_idx(c + 2, ti, ci)

    def pair_body(h, carry):
        step(2 * h, ti1, ci1, ti0, tr1, tr0, ci0, o0)
        step(2 * h + 1, ti0, ci0, ti1, tr0, tr1, ci1, o1)
        return carry

    lax.fori_loop(0, N_CHUNKS // 2, pair_body, 0, unroll=False)

    wait_out(o0)
    wait_out(o1)


@jax.jit
def _launch(tok, cmb, tokw, wkw, hrw, duw):
    mesh = plsc.VectorSubcoreMesh(core_axis_name="c", subcore_axis_name="s")
    kfn = functools.partial(
        pl.kernel,
        mesh=mesh,
        compiler_params=pltpu.CompilerParams(needs_layout_passes=False),
        out_type=jax.ShapeDtypeStruct((B, L, OUT_D), jnp.float32),
        scratch_types=[
            pltpu.VMEM((C,), jnp.int32),
            pltpu.VMEM((C,), jnp.int32),
            pltpu.VMEM((C,), jnp.int32),
            pltpu.VMEM((C,), jnp.int32),
            pltpu.VMEM((7 * 16,), jnp.float32),
            pltpu.VMEM((24 * 16,), jnp.float32),
            pltpu.VMEM((24 * 16,), jnp.float32),
            pltpu.VMEM((C, PAD_D), jnp.float32),
            pltpu.VMEM((C, PAD_D), jnp.float32),
            pltpu.VMEM((C, OUT_D), jnp.float32),
            pltpu.VMEM((C, OUT_D), jnp.float32),
            pltpu.SemaphoreType.DMA,
            pltpu.SemaphoreType.DMA,
            pltpu.SemaphoreType.DMA,
        ],
    )(_body)
    return kfn(tok, cmb, tokw, wkw, hrw, duw)


def kernel(token, week, hour, duration, token_w, week_w, hour_w, dur_w):
    token_w = jnp.pad(token_w, ((0, 0), (0, PAD_D - TOKEN_D)))
    tok = token.reshape(-1).astype(jnp.int32)
    cmb = ((week.astype(jnp.int32) << 22)
           | (hour.astype(jnp.int32) << 13)
           | (duration.astype(jnp.int32) << 4)).reshape(-1)
    return _launch(tok, cmb, token_w,
                   week_w.reshape(-1), hour_w.reshape(-1), dur_w.reshape(-1))
